# baseline (speedup 1.0000x reference)
import jax
import jax.numpy as jnp
from jax.experimental import pallas as pl

B = 8192
EMBED_DIM = 4096
DEPTH = 64
BLK = 512
NSTEPS = B // (2 * BLK)


def _router_body(x0_ref, x1_ref, wt_ref, b_ref, nm_ref, ns_ref,
                 eps_ref, gate_ref, avg_ref, kl_ref):
    i = pl.program_id(0)
    wt = wt_ref[...]
    bias = b_ref[...]
    ns = ns_ref[0, 0]
    nm = nm_ref[0, 0]

    def tile(x_ref, eps, out_slice):
        logits = jnp.dot(x_ref[...], wt, preferred_element_type=jnp.float32)
        logits = logits + bias + (eps * ns + nm)
        m = jnp.max(logits, axis=-1, keepdims=True)
        e = jnp.exp(logits - m)
        s = jnp.sum(e, axis=-1, keepdims=True)
        gate = e / s
        gate_ref[out_slice, :] = gate
        return jnp.sum(gate, axis=0, keepdims=True)

    psum = (tile(x0_ref, eps_ref[:BLK, :], pl.ds(0, BLK))
            + tile(x1_ref, eps_ref[BLK:, :], pl.ds(BLK, BLK)))

    @pl.when(i == 0)
    def _init():
        avg_ref[...] = psum

    @pl.when(i > 0)
    def _acc():
        avg_ref[...] += psum

    @pl.when(i == NSTEPS - 1)
    def _finish():
        ga = avg_ref[...] * (1.0 / B)
        avg_ref[...] = ga
        u = 1.0 / DEPTH
        kl = jnp.sum(u * (jnp.log(u) - jnp.log(ga)),
                     axis=-1, keepdims=True) * (1.0 / DEPTH)
        kl_ref[...] = kl


def kernel(x_embed, W, b, noise_mean, noise_std, eps, train):
    del train
    wt = W.T
    b2 = b.reshape(1, DEPTH)
    nm = noise_mean.reshape(1, 1)
    ns = noise_std.reshape(1, 1)

    gate, gate_avg, kl = pl.pallas_call(
        _router_body,
        grid=(NSTEPS,),
        in_specs=[
            pl.BlockSpec((BLK, EMBED_DIM), lambda i: (2 * i, 0)),
            pl.BlockSpec((BLK, EMBED_DIM), lambda i: (2 * i + 1, 0)),
            pl.BlockSpec((EMBED_DIM, DEPTH), lambda i: (0, 0)),
            pl.BlockSpec((1, DEPTH), lambda i: (0, 0)),
            pl.BlockSpec((1, 1), lambda i: (0, 0)),
            pl.BlockSpec((1, 1), lambda i: (0, 0)),
            pl.BlockSpec((2 * BLK, DEPTH), lambda i: (i, 0)),
        ],
        out_specs=[
            pl.BlockSpec((2 * BLK, DEPTH), lambda i: (i, 0)),
            pl.BlockSpec((1, DEPTH), lambda i: (0, 0)),
            pl.BlockSpec((1, 1), lambda i: (0, 0)),
        ],
        out_shape=[
            jax.ShapeDtypeStruct((B, DEPTH), jnp.float32),
            jax.ShapeDtypeStruct((1, DEPTH), jnp.float32),
            jax.ShapeDtypeStruct((1, 1), jnp.float32),
        ],
    )(x_embed, x_embed, wt, b2, nm, ns, eps)

    return gate, gate_avg.reshape(DEPTH), kl.reshape(())


# speedup vs baseline: 1.0094x; 1.0094x over previous
import jax
import jax.numpy as jnp
from jax.experimental import pallas as pl
from jax.experimental.pallas import tpu as pltpu

B = 8192
EMBED_DIM = 4096
DEPTH = 64
BLK = 512
NSTEPS = B // BLK


def _router_body(x_ref, wt_ref, b_ref, nm_ref, ns_ref, eps_ref,
                 gate_ref, psum_ref):
    logits = jnp.dot(x_ref[...], wt_ref[...],
                     preferred_element_type=jnp.float32)
    logits = logits + b_ref[...] + (eps_ref[...] * ns_ref[0, 0] + nm_ref[0, 0])
    m = jnp.max(logits, axis=-1, keepdims=True)
    e = jnp.exp(logits - m)
    s = jnp.sum(e, axis=-1, keepdims=True)
    gate = e / s
    gate_ref[...] = gate
    psum_ref[...] = jnp.sum(gate, axis=0, keepdims=True)[None]


def _finish_body(psums_ref, avg_ref, kl_ref):
    ga = jnp.sum(psums_ref[...], axis=0) * (1.0 / B)
    avg_ref[...] = ga
    u = 1.0 / DEPTH
    kl = jnp.sum(u * (jnp.log(u) - jnp.log(ga)),
                 axis=-1, keepdims=True) * (1.0 / DEPTH)
    kl_ref[...] = kl


def kernel(x_embed, W, b, noise_mean, noise_std, eps, train):
    del train
    wt = W.T
    b2 = b.reshape(1, DEPTH)
    nm = noise_mean.reshape(1, 1)
    ns = noise_std.reshape(1, 1)

    gate, psums = pl.pallas_call(
        _router_body,
        grid=(NSTEPS,),
        in_specs=[
            pl.BlockSpec((BLK, EMBED_DIM), lambda i: (i, 0)),
            pl.BlockSpec((EMBED_DIM, DEPTH), lambda i: (0, 0)),
            pl.BlockSpec((1, DEPTH), lambda i: (0, 0)),
            pl.BlockSpec((1, 1), lambda i: (0, 0)),
            pl.BlockSpec((1, 1), lambda i: (0, 0)),
            pl.BlockSpec((BLK, DEPTH), lambda i: (i, 0)),
        ],
        out_specs=[
            pl.BlockSpec((BLK, DEPTH), lambda i: (i, 0)),
            pl.BlockSpec((1, 1, DEPTH), lambda i: (i, 0, 0)),
        ],
        out_shape=[
            jax.ShapeDtypeStruct((B, DEPTH), jnp.float32),
            jax.ShapeDtypeStruct((NSTEPS, 1, DEPTH), jnp.float32),
        ],
        compiler_params=pltpu.CompilerParams(
            dimension_semantics=("parallel",)),
    )(x_embed, wt, b2, nm, ns, eps)

    gate_avg, kl = pl.pallas_call(
        _finish_body,
        out_shape=[
            jax.ShapeDtypeStruct((1, DEPTH), jnp.float32),
            jax.ShapeDtypeStruct((1, 1), jnp.float32),
        ],
    )(psums)

    return gate, gate_avg.reshape(DEPTH), kl.reshape(())
